# baseline (speedup 1.0000x reference)
import functools

import jax
import jax.numpy as jnp
from jax import lax
from jax.experimental import pallas as pl
from jax.experimental.pallas import tpu as pltpu
from jax.experimental.pallas import tpu_sc as plsc

_N = 10000
_E = 320000
_NCORES = 2
_NSUB = 16
_NW = _NCORES * _NSUB
_E_PAD = 327680
_EPW0 = 6144
_EPW1 = 14336
_CFG = {
    16: (256, 4, (1, 1)),
    48: (256, 4, (1, 1)),
    64: (256, 4, (1, 2)),
    80: (256, 2, (1, 1)),
    128: (64, 4, (1, 2)),
}
_N_PAD = 10112
_ZROWS = _N_PAD // _NSUB

_HID = 64
_OUT = 40
_F32 = jnp.float32


def _make_prop(d):
  mesh = plsc.VectorSubcoreMesh(core_axis_name="c", subcore_axis_name="s",
                                num_cores=_NCORES, num_subcores=_NSUB)
  k, nbuf, rounds = _CFG[d]
  ch = (_EPW0 // k, _EPW1 // k)
  segs = (ch[0] // rounds[0], ch[1] // rounds[1])
  seg_max = max(segs)
  for s in segs:
    assert s % nbuf == 0

  def body(h_hbm, src_hbm, dst_hbm, zero_hbm, out_hbm,
           src_i, dst_i, rows_v, acc, gsems, ssems):
    cid = lax.axis_index("c")
    sid = lax.axis_index("s")
    pltpu.sync_copy(zero_hbm.at[pl.ds(sid * _ZROWS, _ZROWS)],
                    acc.at[pl.ds(sid * _ZROWS, _ZROWS)])
    plsc.subcore_barrier()

    def run(base_chunk, n_rounds, seg):
      for r in range(n_rounds):
        roff = base_chunk + r * seg
        pltpu.sync_copy(src_hbm.at[pl.ds(roff, seg)],
                        src_i.at[pl.ds(0, seg)])
        pltpu.sync_copy(dst_hbm.at[pl.ds(roff, seg)],
                        dst_i.at[pl.ds(0, seg)])
        for b in range(nbuf - 1):
          pltpu.async_copy(h_hbm.at[src_i.at[b]], rows_v.at[b], gsems.at[b])

        def step(g, carry):
          for b in range(nbuf):
            j = nbuf * g + b
            bn = (b + nbuf - 1) % nbuf
            pltpu.make_async_copy(h_hbm.at[src_i.at[j]], rows_v.at[b],
                                  gsems.at[b]).wait()
            pltpu.async_copy(rows_v.at[b], acc.at[dst_i.at[j]], ssems.at[b],
                             add=True)

            @pl.when(j >= 1)
            def _():
              pltpu.make_async_copy(rows_v.at[bn],
                                    acc.at[dst_i.at[jnp.maximum(j - 1, 0)]],
                                    ssems.at[bn]).wait()

            @pl.when(j + nbuf - 1 < seg)
            def _():
              pltpu.async_copy(h_hbm.at[src_i.at[j + nbuf - 1]],
                               rows_v.at[bn], gsems.at[bn])
          return carry

        lax.fori_loop(0, seg // nbuf, step, 0)
        pltpu.make_async_copy(rows_v.at[(seg - 1) % nbuf],
                              acc.at[dst_i.at[seg - 1]],
                              ssems.at[(seg - 1) % nbuf]).wait()

    @pl.when(cid == 0)
    def _():
      run(sid * ch[0], rounds[0], segs[0])

    @pl.when(cid == 1)
    def _():
      run(_NSUB * ch[0] + sid * ch[1], rounds[1], segs[1])

    plsc.subcore_barrier()
    pltpu.sync_copy(acc.at[pl.ds(sid * _ZROWS, _ZROWS)],
                    out_hbm.at[cid, pl.ds(sid * _ZROWS, _ZROWS)])

  return pl.kernel(
      body,
      out_type=jax.ShapeDtypeStruct((_NCORES, _N_PAD, d), _F32),
      mesh=mesh,
      compiler_params=pltpu.CompilerParams(use_tc_tiling_on_sc=False),
      scratch_types=[
          pltpu.VMEM((seg_max, k), jnp.int32),
          pltpu.VMEM((seg_max, k), jnp.int32),
          pltpu.VMEM((nbuf, k, d), _F32),
          pltpu.VMEM_SHARED((_N_PAD, d), _F32),
          pltpu.SemaphoreType.DMA((nbuf,)),
          pltpu.SemaphoreType.DMA((nbuf,)),
      ],
  )


def _make_deg():
  d, k, nbuf = 16, 256, 4
  ch = (_EPW0 // k, _EPW1 // k)
  mesh = plsc.VectorSubcoreMesh(core_axis_name="c", subcore_axis_name="s",
                                num_cores=_NCORES, num_subcores=_NSUB)

  def body(ones_hbm, dst_hbm, zero_hbm, out_hbm, dst_i, ones_v, acc,
           gsem, ssems):
    cid = lax.axis_index("c")
    sid = lax.axis_index("s")
    pltpu.sync_copy(zero_hbm.at[pl.ds(sid * _ZROWS, _ZROWS)],
                    acc.at[pl.ds(sid * _ZROWS, _ZROWS)])
    pltpu.async_copy(ones_hbm, ones_v, gsem).wait()
    plsc.subcore_barrier()

    def run(base_chunk, chunks):
      pltpu.sync_copy(dst_hbm.at[pl.ds(base_chunk, chunks)],
                      dst_i.at[pl.ds(0, chunks)])

      def step(g, carry):
        for b in range(nbuf):
          j = nbuf * g + b

          @pl.when(j >= nbuf)
          def _():
            pltpu.make_async_copy(
                ones_v, acc.at[dst_i.at[jnp.maximum(j - nbuf, 0)]],
                ssems.at[b]).wait()

          pltpu.async_copy(ones_v, acc.at[dst_i.at[j]], ssems.at[b], add=True)
        return carry

      lax.fori_loop(0, chunks // nbuf, step, 0)
      for b in range(nbuf):
        pltpu.make_async_copy(ones_v, acc.at[dst_i.at[chunks - nbuf + b]],
                              ssems.at[b]).wait()

    @pl.when(cid == 0)
    def _():
      run(sid * ch[0], ch[0])

    @pl.when(cid == 1)
    def _():
      run(_NSUB * ch[0] + sid * ch[1], ch[1])

    plsc.subcore_barrier()
    pltpu.sync_copy(acc.at[pl.ds(sid * _ZROWS, _ZROWS)],
                    out_hbm.at[cid, pl.ds(sid * _ZROWS, _ZROWS)])

  return pl.kernel(
      body,
      out_type=jax.ShapeDtypeStruct((_NCORES, _N_PAD, d), _F32),
      mesh=mesh,
      compiler_params=pltpu.CompilerParams(use_tc_tiling_on_sc=False),
      scratch_types=[
          pltpu.VMEM((max(ch), k), jnp.int32),
          pltpu.VMEM((k, d), _F32),
          pltpu.VMEM_SHARED((_N_PAD, d), _F32),
          pltpu.SemaphoreType.DMA,
          pltpu.SemaphoreType.DMA((nbuf,)),
      ],
  )


_prop_cache = {}


def _deg(dst_flat, zeros):
  if "deg" not in _prop_cache:
    _prop_cache["deg"] = _make_deg()
  k = 256
  ones = jnp.ones((k, 16), _F32)
  dst2 = dst_flat.reshape(_E_PAD // k, k)
  return _prop_cache["deg"](ones, dst2, zeros)


def _prop(d, h, src_flat, dst_flat, zeros):
  if d not in _prop_cache:
    _prop_cache[d] = _make_prop(d)
  k = _CFG[d][0]
  src2 = src_flat.reshape(_E_PAD // k, k)
  dst2 = dst_flat.reshape(_E_PAD // k, k)
  return _prop_cache[d](h, src2, dst2, zeros)


_BM = 512
_G = (_N + _BM - 1) // _BM


def _row_spec(d):
  return pl.BlockSpec((_BM, d), lambda i: (i, 0))


def _pair_spec(d):
  return pl.BlockSpec((_NCORES, _BM, d), lambda i: (0, i, 0))


def _full_spec(r, c):
  return pl.BlockSpec((r, c), lambda i: (0, 0))


def _prep0_body(deg_ref, x_ref, w_ref, p0_ref, g_ref, n_ref):
  deg = deg_ref[0, :, 0] + deg_ref[1, :, 0]
  norm = jnp.where(deg > 0, lax.rsqrt(jnp.maximum(deg, 1e-12)), 0.0)
  n = norm[:, None]
  n_ref[...] = n
  h = jnp.dot(x_ref[...], w_ref[...], preferred_element_type=_F32)
  p0_ref[...] = h[:, :_HID]
  g_ref[...] = h[:, _HID:] * n


_prep0 = pl.pallas_call(
    _prep0_body,
    grid=(_G,),
    in_specs=[_pair_spec(16), _row_spec(128), _full_spec(128, 3 * _HID)],
    out_specs=[_row_spec(_HID), _row_spec(2 * _HID), _row_spec(1)],
    out_shape=[
        jax.ShapeDtypeStruct((_N, _HID), _F32),
        jax.ShapeDtypeStruct((_N, 2 * _HID), _F32),
        jax.ShapeDtypeStruct((_N, 1), _F32),
    ],
)


def _comb_body(split, pad, s_ref, n_ref, a_ref, b_ref):
  s = s_ref[0] + s_ref[1]
  n = n_ref[...]
  a_ref[...] = s[:, :split] * n
  b = s[:, split:] * (n * n)
  if pad:
    b = jnp.concatenate([b, jnp.zeros((_BM, pad), _F32)], axis=1)
  b_ref[...] = b


def _make_comb(d, split, pad):
  return pl.pallas_call(
      functools.partial(_comb_body, split, pad),
      grid=(_G,),
      in_specs=[_pair_spec(d), _row_spec(1)],
      out_specs=[_row_spec(split), _row_spec(d - split + pad)],
      out_shape=[
          jax.ShapeDtypeStruct((_N, split), _F32),
          jax.ShapeDtypeStruct((_N, d - split + pad), _F32),
      ],
  )


_comb0 = _make_comb(2 * _HID, _HID, 0)
_comb1 = _make_comb(2 * _OUT, _OUT, 8)


def _hid_body(p0_ref, a1_ref, s2_ref, n_ref, w_ref, q0_ref, u_ref):
  n = n_ref[...]
  a2 = (s2_ref[0] + s2_ref[1]) * n
  hidden = jnp.concatenate([p0_ref[...], a1_ref[...], a2], axis=1)
  hidden = jnp.maximum(hidden, 0.0)
  h1 = jnp.dot(hidden, w_ref[...], preferred_element_type=_F32)
  q0_ref[...] = h1[:, :_OUT]
  u_ref[...] = h1[:, _OUT:] * n


_hid = pl.pallas_call(
    _hid_body,
    grid=(_G,),
    in_specs=[_row_spec(_HID), _row_spec(_HID), _pair_spec(_HID),
              _row_spec(1), _full_spec(3 * _HID, 3 * _OUT)],
    out_specs=[_row_spec(_OUT), _row_spec(2 * _OUT)],
    out_shape=[
        jax.ShapeDtypeStruct((_N, _OUT), _F32),
        jax.ShapeDtypeStruct((_N, 2 * _OUT), _F32),
    ],
)


def _fin_body(q0_ref, au1_ref, s4_ref, n_ref, o_ref):
  n = n_ref[...]
  a2u2 = (s4_ref[0] + s4_ref[1])[:, :_OUT] * n
  z = jnp.concatenate([q0_ref[...], au1_ref[...], a2u2], axis=1)
  m = jnp.max(z, axis=1, keepdims=True)
  lse = jnp.log(jnp.sum(jnp.exp(z - m), axis=1, keepdims=True)) + m
  o_ref[...] = z - lse


_fin = pl.pallas_call(
    _fin_body,
    grid=(_G,),
    in_specs=[_row_spec(_OUT), _row_spec(_OUT), _pair_spec(48), _row_spec(1)],
    out_specs=_row_spec(3 * _OUT),
    out_shape=jax.ShapeDtypeStruct((_N, 3 * _OUT), _F32),
)


def kernel(x, edge_index, W0_0, W0_1, W0_2, W1_0, W1_1, W1_2):
  src = edge_index[0]
  dst = edge_index[1]
  pad = _E_PAD - _E
  src_p = jnp.concatenate([src, jnp.zeros((pad,), jnp.int32)])
  dst_p = jnp.concatenate([dst, jnp.full((pad,), _N, jnp.int32)])

  z16 = jnp.zeros((_N_PAD, 16), _F32)
  z128 = jnp.zeros((_N_PAD, 128), _F32)
  z64 = jnp.zeros((_N_PAD, 64), _F32)
  z80 = jnp.zeros((_N_PAD, 80), _F32)
  z48 = jnp.zeros((_N_PAD, 48), _F32)

  degacc = _deg(dst_p, z16)

  W0c = jnp.concatenate([W0_0, W0_1, W0_2], axis=1)
  p0, g, norm = _prep0(degacc, x, W0c)

  s1 = _prop(128, g, src_p, dst_p, z128)
  at1, mid = _comb0(s1, norm)
  s2 = _prop(64, mid, src_p, dst_p, z64)

  W1c = jnp.concatenate([W1_0, W1_1, W1_2], axis=1)
  q0, u = _hid(p0, at1, s2, norm, W1c)

  s3 = _prop(80, u, src_p, dst_p, z80)
  au1, mid2 = _comb1(s3, norm)
  s4 = _prop(48, mid2, src_p, dst_p, z48)

  return _fin(q0, au1, s4, norm)


# speedup vs baseline: 1.0879x; 1.0879x over previous
import functools

import jax
import jax.numpy as jnp
from jax import lax
from jax.experimental import pallas as pl
from jax.experimental.pallas import tpu as pltpu
from jax.experimental.pallas import tpu_sc as plsc

_N = 10000
_E = 320000
_NCORES = 2
_NSUB = 16
_NW = _NCORES * _NSUB
_E_PAD = 327680
_EPW0 = 14336
_EPW1 = 6144
_CFG = {
    16: (256, 4, (1, 1)),
    48: (256, 4, (1, 1)),
    64: (256, 4, (2, 1)),
    80: (256, 2, (1, 1)),
    128: (64, 4, (2, 1)),
}
_N_PAD = 10112
_ZROWS = _N_PAD // _NSUB

_HID = 64
_OUT = 40
_F32 = jnp.float32


def _make_prop(d):
  mesh = plsc.VectorSubcoreMesh(core_axis_name="c", subcore_axis_name="s",
                                num_cores=_NCORES, num_subcores=_NSUB)
  k, nbuf, rounds = _CFG[d]
  ch = (_EPW0 // k, _EPW1 // k)
  segs = (ch[0] // rounds[0], ch[1] // rounds[1])
  seg_max = max(segs)
  for s in segs:
    assert s % nbuf == 0

  def body(h_hbm, src_hbm, dst_hbm, zero_hbm, out_hbm,
           src_i, dst_i, rows_v, acc, gsems, ssems):
    cid = lax.axis_index("c")
    sid = lax.axis_index("s")
    pltpu.sync_copy(zero_hbm.at[pl.ds(sid * _ZROWS, _ZROWS)],
                    acc.at[pl.ds(sid * _ZROWS, _ZROWS)])
    plsc.subcore_barrier()

    def run(base_chunk, n_rounds, seg):
      for r in range(n_rounds):
        roff = base_chunk + r * seg
        pltpu.sync_copy(src_hbm.at[pl.ds(roff, seg)],
                        src_i.at[pl.ds(0, seg)])
        pltpu.sync_copy(dst_hbm.at[pl.ds(roff, seg)],
                        dst_i.at[pl.ds(0, seg)])
        for b in range(nbuf - 1):
          pltpu.async_copy(h_hbm.at[src_i.at[b]], rows_v.at[b], gsems.at[b])

        def step(g, carry):
          for b in range(nbuf):
            j = nbuf * g + b
            bn = (b + nbuf - 1) % nbuf
            pltpu.make_async_copy(h_hbm.at[src_i.at[j]], rows_v.at[b],
                                  gsems.at[b]).wait()
            pltpu.async_copy(rows_v.at[b], acc.at[dst_i.at[j]], ssems.at[b],
                             add=True)

            @pl.when(j >= 1)
            def _():
              pltpu.make_async_copy(rows_v.at[bn],
                                    acc.at[dst_i.at[jnp.maximum(j - 1, 0)]],
                                    ssems.at[bn]).wait()

            @pl.when(j + nbuf - 1 < seg)
            def _():
              pltpu.async_copy(h_hbm.at[src_i.at[j + nbuf - 1]],
                               rows_v.at[bn], gsems.at[bn])
          return carry

        lax.fori_loop(0, seg // nbuf, step, 0)
        pltpu.make_async_copy(rows_v.at[(seg - 1) % nbuf],
                              acc.at[dst_i.at[seg - 1]],
                              ssems.at[(seg - 1) % nbuf]).wait()

    @pl.when(cid == 0)
    def _():
      run(sid * ch[0], rounds[0], segs[0])

    @pl.when(cid == 1)
    def _():
      run(_NSUB * ch[0] + sid * ch[1], rounds[1], segs[1])

    plsc.subcore_barrier()
    pltpu.sync_copy(acc.at[pl.ds(sid * _ZROWS, _ZROWS)],
                    out_hbm.at[cid, pl.ds(sid * _ZROWS, _ZROWS)])

  return pl.kernel(
      body,
      out_type=jax.ShapeDtypeStruct((_NCORES, _N_PAD, d), _F32),
      mesh=mesh,
      compiler_params=pltpu.CompilerParams(use_tc_tiling_on_sc=False),
      scratch_types=[
          pltpu.VMEM((seg_max, k), jnp.int32),
          pltpu.VMEM((seg_max, k), jnp.int32),
          pltpu.VMEM((nbuf, k, d), _F32),
          pltpu.VMEM_SHARED((_N_PAD, d), _F32),
          pltpu.SemaphoreType.DMA((nbuf,)),
          pltpu.SemaphoreType.DMA((nbuf,)),
      ],
  )


def _make_deg():
  d, k, nbuf = 16, 256, 4
  ch = (_EPW0 // k, _EPW1 // k)
  mesh = plsc.VectorSubcoreMesh(core_axis_name="c", subcore_axis_name="s",
                                num_cores=_NCORES, num_subcores=_NSUB)

  def body(ones_hbm, dst_hbm, zero_hbm, out_hbm, dst_i, ones_v, acc,
           gsem, ssems):
    cid = lax.axis_index("c")
    sid = lax.axis_index("s")
    pltpu.sync_copy(zero_hbm.at[pl.ds(sid * _ZROWS, _ZROWS)],
                    acc.at[pl.ds(sid * _ZROWS, _ZROWS)])
    pltpu.async_copy(ones_hbm, ones_v, gsem).wait()
    plsc.subcore_barrier()

    def run(base_chunk, chunks):
      pltpu.sync_copy(dst_hbm.at[pl.ds(base_chunk, chunks)],
                      dst_i.at[pl.ds(0, chunks)])

      def step(g, carry):
        for b in range(nbuf):
          j = nbuf * g + b

          @pl.when(j >= nbuf)
          def _():
            pltpu.make_async_copy(
                ones_v, acc.at[dst_i.at[jnp.maximum(j - nbuf, 0)]],
                ssems.at[b]).wait()

          pltpu.async_copy(ones_v, acc.at[dst_i.at[j]], ssems.at[b], add=True)
        return carry

      lax.fori_loop(0, chunks // nbuf, step, 0)
      for b in range(nbuf):
        pltpu.make_async_copy(ones_v, acc.at[dst_i.at[chunks - nbuf + b]],
                              ssems.at[b]).wait()

    @pl.when(cid == 0)
    def _():
      run(sid * ch[0], ch[0])

    @pl.when(cid == 1)
    def _():
      run(_NSUB * ch[0] + sid * ch[1], ch[1])

    plsc.subcore_barrier()
    pltpu.sync_copy(acc.at[pl.ds(sid * _ZROWS, _ZROWS)],
                    out_hbm.at[cid, pl.ds(sid * _ZROWS, _ZROWS)])

  return pl.kernel(
      body,
      out_type=jax.ShapeDtypeStruct((_NCORES, _N_PAD, d), _F32),
      mesh=mesh,
      compiler_params=pltpu.CompilerParams(use_tc_tiling_on_sc=False),
      scratch_types=[
          pltpu.VMEM((max(ch), k), jnp.int32),
          pltpu.VMEM((k, d), _F32),
          pltpu.VMEM_SHARED((_N_PAD, d), _F32),
          pltpu.SemaphoreType.DMA,
          pltpu.SemaphoreType.DMA((nbuf,)),
      ],
  )


_prop_cache = {}


def _deg(dst_flat, zeros):
  if "deg" not in _prop_cache:
    _prop_cache["deg"] = _make_deg()
  k = 256
  ones = jnp.ones((k, 16), _F32)
  dst2 = dst_flat.reshape(_E_PAD // k, k)
  return _prop_cache["deg"](ones, dst2, zeros)


def _prop(d, h, src_flat, dst_flat, zeros):
  if d not in _prop_cache:
    _prop_cache[d] = _make_prop(d)
  k = _CFG[d][0]
  src2 = src_flat.reshape(_E_PAD // k, k)
  dst2 = dst_flat.reshape(_E_PAD // k, k)
  return _prop_cache[d](h, src2, dst2, zeros)


_BM = 512
_G = (_N + _BM - 1) // _BM


def _row_spec(d):
  return pl.BlockSpec((_BM, d), lambda i: (i, 0))


def _pair_spec(d):
  return pl.BlockSpec((_NCORES, _BM, d), lambda i: (0, i, 0))


def _full_spec(r, c):
  return pl.BlockSpec((r, c), lambda i: (0, 0))


def _prep0_body(deg_ref, x_ref, w_ref, p0_ref, g_ref, n_ref):
  deg = deg_ref[0, :, 0] + deg_ref[1, :, 0]
  norm = jnp.where(deg > 0, lax.rsqrt(jnp.maximum(deg, 1e-12)), 0.0)
  n = norm[:, None]
  n_ref[...] = n
  h = jnp.dot(x_ref[...], w_ref[...], preferred_element_type=_F32)
  p0_ref[...] = h[:, :_HID]
  g_ref[...] = h[:, _HID:] * n


_prep0 = pl.pallas_call(
    _prep0_body,
    grid=(_G,),
    in_specs=[_pair_spec(16), _row_spec(128), _full_spec(128, 3 * _HID)],
    out_specs=[_row_spec(_HID), _row_spec(2 * _HID), _row_spec(1)],
    out_shape=[
        jax.ShapeDtypeStruct((_N, _HID), _F32),
        jax.ShapeDtypeStruct((_N, 2 * _HID), _F32),
        jax.ShapeDtypeStruct((_N, 1), _F32),
    ],
)


def _comb_body(split, pad, s_ref, n_ref, a_ref, b_ref):
  s = s_ref[0] + s_ref[1]
  n = n_ref[...]
  a_ref[...] = s[:, :split] * n
  b = s[:, split:] * (n * n)
  if pad:
    b = jnp.concatenate([b, jnp.zeros((_BM, pad), _F32)], axis=1)
  b_ref[...] = b


def _make_comb(d, split, pad):
  return pl.pallas_call(
      functools.partial(_comb_body, split, pad),
      grid=(_G,),
      in_specs=[_pair_spec(d), _row_spec(1)],
      out_specs=[_row_spec(split), _row_spec(d - split + pad)],
      out_shape=[
          jax.ShapeDtypeStruct((_N, split), _F32),
          jax.ShapeDtypeStruct((_N, d - split + pad), _F32),
      ],
  )


_comb0 = _make_comb(2 * _HID, _HID, 0)
_comb1 = _make_comb(2 * _OUT, _OUT, 8)


def _hid_body(p0_ref, a1_ref, s2_ref, n_ref, w_ref, q0_ref, u_ref):
  n = n_ref[...]
  a2 = (s2_ref[0] + s2_ref[1]) * n
  hidden = jnp.concatenate([p0_ref[...], a1_ref[...], a2], axis=1)
  hidden = jnp.maximum(hidden, 0.0)
  h1 = jnp.dot(hidden, w_ref[...], preferred_element_type=_F32)
  q0_ref[...] = h1[:, :_OUT]
  u_ref[...] = h1[:, _OUT:] * n


_hid = pl.pallas_call(
    _hid_body,
    grid=(_G,),
    in_specs=[_row_spec(_HID), _row_spec(_HID), _pair_spec(_HID),
              _row_spec(1), _full_spec(3 * _HID, 3 * _OUT)],
    out_specs=[_row_spec(_OUT), _row_spec(2 * _OUT)],
    out_shape=[
        jax.ShapeDtypeStruct((_N, _OUT), _F32),
        jax.ShapeDtypeStruct((_N, 2 * _OUT), _F32),
    ],
)


def _fin_body(q0_ref, au1_ref, s4_ref, n_ref, o_ref):
  n = n_ref[...]
  a2u2 = (s4_ref[0] + s4_ref[1])[:, :_OUT] * n
  z = jnp.concatenate([q0_ref[...], au1_ref[...], a2u2], axis=1)
  m = jnp.max(z, axis=1, keepdims=True)
  lse = jnp.log(jnp.sum(jnp.exp(z - m), axis=1, keepdims=True)) + m
  o_ref[...] = z - lse


_fin = pl.pallas_call(
    _fin_body,
    grid=(_G,),
    in_specs=[_row_spec(_OUT), _row_spec(_OUT), _pair_spec(48), _row_spec(1)],
    out_specs=_row_spec(3 * _OUT),
    out_shape=jax.ShapeDtypeStruct((_N, 3 * _OUT), _F32),
)


def kernel(x, edge_index, W0_0, W0_1, W0_2, W1_0, W1_1, W1_2):
  src = edge_index[0]
  dst = edge_index[1]
  pad = _E_PAD - _E
  src_p = jnp.concatenate([src, jnp.zeros((pad,), jnp.int32)])
  dst_p = jnp.concatenate([dst, jnp.full((pad,), _N, jnp.int32)])

  z16 = jnp.zeros((_N_PAD, 16), _F32)
  z128 = jnp.zeros((_N_PAD, 128), _F32)
  z64 = jnp.zeros((_N_PAD, 64), _F32)
  z80 = jnp.zeros((_N_PAD, 80), _F32)
  z48 = jnp.zeros((_N_PAD, 48), _F32)

  degacc = _deg(dst_p, z16)

  W0c = jnp.concatenate([W0_0, W0_1, W0_2], axis=1)
  p0, g, norm = _prep0(degacc, x, W0c)

  s1 = _prop(128, g, src_p, dst_p, z128)
  at1, mid = _comb0(s1, norm)
  s2 = _prop(64, mid, src_p, dst_p, z64)

  W1c = jnp.concatenate([W1_0, W1_1, W1_2], axis=1)
  q0, u = _hid(p0, at1, s2, norm, W1c)

  s3 = _prop(80, u, src_p, dst_p, z80)
  au1, mid2 = _comb1(s3, norm)
  s4 = _prop(48, mid2, src_p, dst_p, z48)

  return _fin(q0, au1, s4, norm)


# speedup vs baseline: 1.1040x; 1.0148x over previous
import functools

import jax
import jax.numpy as jnp
from jax import lax
from jax.experimental import pallas as pl
from jax.experimental.pallas import tpu as pltpu
from jax.experimental.pallas import tpu_sc as plsc

_N = 10000
_E = 320000
_NCORES = 2
_NSUB = 16
_NW = _NCORES * _NSUB
_E_PAD = 327680
_EPW0 = 16384
_EPW1 = 4096
_CFG = {
    16: (256, 4, (1, 1)),
    48: (256, 4, (1, 1)),
    64: (256, 4, (2, 1)),
    80: (256, 2, (1, 1)),
    128: (64, 4, (2, 1)),
}
_N_PAD = 10112
_ZROWS = _N_PAD // _NSUB

_HID = 64
_OUT = 40
_F32 = jnp.float32


def _make_prop(d):
  mesh = plsc.VectorSubcoreMesh(core_axis_name="c", subcore_axis_name="s",
                                num_cores=_NCORES, num_subcores=_NSUB)
  k, nbuf, rounds = _CFG[d]
  ch = (_EPW0 // k, _EPW1 // k)
  segs = (ch[0] // rounds[0], ch[1] // rounds[1])
  seg_max = max(segs)
  for s in segs:
    assert s % nbuf == 0

  def body(h_hbm, src_hbm, dst_hbm, zero_hbm, out_hbm,
           src_i, dst_i, rows_v, acc, gsems, ssems):
    cid = lax.axis_index("c")
    sid = lax.axis_index("s")
    pltpu.sync_copy(zero_hbm.at[pl.ds(sid * _ZROWS, _ZROWS)],
                    acc.at[pl.ds(sid * _ZROWS, _ZROWS)])
    plsc.subcore_barrier()

    def run(base_chunk, n_rounds, seg):
      for r in range(n_rounds):
        roff = base_chunk + r * seg
        pltpu.sync_copy(src_hbm.at[pl.ds(roff, seg)],
                        src_i.at[pl.ds(0, seg)])
        pltpu.sync_copy(dst_hbm.at[pl.ds(roff, seg)],
                        dst_i.at[pl.ds(0, seg)])
        for b in range(nbuf - 1):
          pltpu.async_copy(h_hbm.at[src_i.at[b]], rows_v.at[b], gsems.at[b])

        def step(g, carry):
          for b in range(nbuf):
            j = nbuf * g + b
            bn = (b + nbuf - 1) % nbuf
            pltpu.make_async_copy(h_hbm.at[src_i.at[j]], rows_v.at[b],
                                  gsems.at[b]).wait()
            pltpu.async_copy(rows_v.at[b], acc.at[dst_i.at[j]], ssems.at[b],
                             add=True)

            @pl.when(j >= 1)
            def _():
              pltpu.make_async_copy(rows_v.at[bn],
                                    acc.at[dst_i.at[jnp.maximum(j - 1, 0)]],
                                    ssems.at[bn]).wait()

            @pl.when(j + nbuf - 1 < seg)
            def _():
              pltpu.async_copy(h_hbm.at[src_i.at[j + nbuf - 1]],
                               rows_v.at[bn], gsems.at[bn])
          return carry

        lax.fori_loop(0, seg // nbuf, step, 0)
        pltpu.make_async_copy(rows_v.at[(seg - 1) % nbuf],
                              acc.at[dst_i.at[seg - 1]],
                              ssems.at[(seg - 1) % nbuf]).wait()

    @pl.when(cid == 0)
    def _():
      run(sid * ch[0], rounds[0], segs[0])

    @pl.when(cid == 1)
    def _():
      run(_NSUB * ch[0] + sid * ch[1], rounds[1], segs[1])

    plsc.subcore_barrier()
    pltpu.sync_copy(acc.at[pl.ds(sid * _ZROWS, _ZROWS)],
                    out_hbm.at[cid, pl.ds(sid * _ZROWS, _ZROWS)])

  return pl.kernel(
      body,
      out_type=jax.ShapeDtypeStruct((_NCORES, _N_PAD, d), _F32),
      mesh=mesh,
      compiler_params=pltpu.CompilerParams(use_tc_tiling_on_sc=False),
      scratch_types=[
          pltpu.VMEM((seg_max, k), jnp.int32),
          pltpu.VMEM((seg_max, k), jnp.int32),
          pltpu.VMEM((nbuf, k, d), _F32),
          pltpu.VMEM_SHARED((_N_PAD, d), _F32),
          pltpu.SemaphoreType.DMA((nbuf,)),
          pltpu.SemaphoreType.DMA((nbuf,)),
      ],
  )


def _make_deg():
  d, k, nbuf = 16, 256, 4
  ch = (_EPW0 // k, _EPW1 // k)
  mesh = plsc.VectorSubcoreMesh(core_axis_name="c", subcore_axis_name="s",
                                num_cores=_NCORES, num_subcores=_NSUB)

  def body(ones_hbm, dst_hbm, zero_hbm, out_hbm, dst_i, ones_v, acc,
           gsem, ssems):
    cid = lax.axis_index("c")
    sid = lax.axis_index("s")
    pltpu.sync_copy(zero_hbm.at[pl.ds(sid * _ZROWS, _ZROWS)],
                    acc.at[pl.ds(sid * _ZROWS, _ZROWS)])
    pltpu.async_copy(ones_hbm, ones_v, gsem).wait()
    plsc.subcore_barrier()

    def run(base_chunk, chunks):
      pltpu.sync_copy(dst_hbm.at[pl.ds(base_chunk, chunks)],
                      dst_i.at[pl.ds(0, chunks)])

      def step(g, carry):
        for b in range(nbuf):
          j = nbuf * g + b

          @pl.when(j >= nbuf)
          def _():
            pltpu.make_async_copy(
                ones_v, acc.at[dst_i.at[jnp.maximum(j - nbuf, 0)]],
                ssems.at[b]).wait()

          pltpu.async_copy(ones_v, acc.at[dst_i.at[j]], ssems.at[b], add=True)
        return carry

      lax.fori_loop(0, chunks // nbuf, step, 0)
      for b in range(nbuf):
        pltpu.make_async_copy(ones_v, acc.at[dst_i.at[chunks - nbuf + b]],
                              ssems.at[b]).wait()

    @pl.when(cid == 0)
    def _():
      run(sid * ch[0], ch[0])

    @pl.when(cid == 1)
    def _():
      run(_NSUB * ch[0] + sid * ch[1], ch[1])

    plsc.subcore_barrier()
    pltpu.sync_copy(acc.at[pl.ds(sid * _ZROWS, _ZROWS)],
                    out_hbm.at[cid, pl.ds(sid * _ZROWS, _ZROWS)])

  return pl.kernel(
      body,
      out_type=jax.ShapeDtypeStruct((_NCORES, _N_PAD, d), _F32),
      mesh=mesh,
      compiler_params=pltpu.CompilerParams(use_tc_tiling_on_sc=False),
      scratch_types=[
          pltpu.VMEM((max(ch), k), jnp.int32),
          pltpu.VMEM((k, d), _F32),
          pltpu.VMEM_SHARED((_N_PAD, d), _F32),
          pltpu.SemaphoreType.DMA,
          pltpu.SemaphoreType.DMA((nbuf,)),
      ],
  )


_prop_cache = {}


def _deg(dst_flat, zeros):
  if "deg" not in _prop_cache:
    _prop_cache["deg"] = _make_deg()
  k = 256
  ones = jnp.ones((k, 16), _F32)
  dst2 = dst_flat.reshape(_E_PAD // k, k)
  return _prop_cache["deg"](ones, dst2, zeros)


def _prop(d, h, src_flat, dst_flat, zeros):
  if d not in _prop_cache:
    _prop_cache[d] = _make_prop(d)
  k = _CFG[d][0]
  src2 = src_flat.reshape(_E_PAD // k, k)
  dst2 = dst_flat.reshape(_E_PAD // k, k)
  return _prop_cache[d](h, src2, dst2, zeros)


_BM = 512
_G = (_N + _BM - 1) // _BM


def _row_spec(d):
  return pl.BlockSpec((_BM, d), lambda i: (i, 0))


def _pair_spec(d):
  return pl.BlockSpec((_NCORES, _BM, d), lambda i: (0, i, 0))


def _full_spec(r, c):
  return pl.BlockSpec((r, c), lambda i: (0, 0))


def _prep0_body(deg_ref, x_ref, w_ref, p0_ref, g_ref, n_ref):
  deg = deg_ref[0, :, 0] + deg_ref[1, :, 0]
  norm = jnp.where(deg > 0, lax.rsqrt(jnp.maximum(deg, 1e-12)), 0.0)
  n = norm[:, None]
  n_ref[...] = n
  h = jnp.dot(x_ref[...], w_ref[...], preferred_element_type=_F32)
  p0_ref[...] = h[:, :_HID]
  g_ref[...] = h[:, _HID:] * n


_prep0 = pl.pallas_call(
    _prep0_body,
    grid=(_G,),
    in_specs=[_pair_spec(16), _row_spec(128), _full_spec(128, 3 * _HID)],
    out_specs=[_row_spec(_HID), _row_spec(2 * _HID), _row_spec(1)],
    out_shape=[
        jax.ShapeDtypeStruct((_N, _HID), _F32),
        jax.ShapeDtypeStruct((_N, 2 * _HID), _F32),
        jax.ShapeDtypeStruct((_N, 1), _F32),
    ],
)


def _comb_body(split, pad, s_ref, n_ref, a_ref, b_ref):
  s = s_ref[0] + s_ref[1]
  n = n_ref[...]
  a_ref[...] = s[:, :split] * n
  b = s[:, split:] * (n * n)
  if pad:
    b = jnp.concatenate([b, jnp.zeros((_BM, pad), _F32)], axis=1)
  b_ref[...] = b


def _make_comb(d, split, pad):
  return pl.pallas_call(
      functools.partial(_comb_body, split, pad),
      grid=(_G,),
      in_specs=[_pair_spec(d), _row_spec(1)],
      out_specs=[_row_spec(split), _row_spec(d - split + pad)],
      out_shape=[
          jax.ShapeDtypeStruct((_N, split), _F32),
          jax.ShapeDtypeStruct((_N, d - split + pad), _F32),
      ],
  )


_comb0 = _make_comb(2 * _HID, _HID, 0)
_comb1 = _make_comb(2 * _OUT, _OUT, 8)


def _hid_body(p0_ref, a1_ref, s2_ref, n_ref, w_ref, q0_ref, u_ref):
  n = n_ref[...]
  a2 = (s2_ref[0] + s2_ref[1]) * n
  hidden = jnp.concatenate([p0_ref[...], a1_ref[...], a2], axis=1)
  hidden = jnp.maximum(hidden, 0.0)
  h1 = jnp.dot(hidden, w_ref[...], preferred_element_type=_F32)
  q0_ref[...] = h1[:, :_OUT]
  u_ref[...] = h1[:, _OUT:] * n


_hid = pl.pallas_call(
    _hid_body,
    grid=(_G,),
    in_specs=[_row_spec(_HID), _row_spec(_HID), _pair_spec(_HID),
              _row_spec(1), _full_spec(3 * _HID, 3 * _OUT)],
    out_specs=[_row_spec(_OUT), _row_spec(2 * _OUT)],
    out_shape=[
        jax.ShapeDtypeStruct((_N, _OUT), _F32),
        jax.ShapeDtypeStruct((_N, 2 * _OUT), _F32),
    ],
)


def _fin_body(q0_ref, au1_ref, s4_ref, n_ref, o_ref):
  n = n_ref[...]
  a2u2 = (s4_ref[0] + s4_ref[1])[:, :_OUT] * n
  z = jnp.concatenate([q0_ref[...], au1_ref[...], a2u2], axis=1)
  m = jnp.max(z, axis=1, keepdims=True)
  lse = jnp.log(jnp.sum(jnp.exp(z - m), axis=1, keepdims=True)) + m
  o_ref[...] = z - lse


_fin = pl.pallas_call(
    _fin_body,
    grid=(_G,),
    in_specs=[_row_spec(_OUT), _row_spec(_OUT), _pair_spec(48), _row_spec(1)],
    out_specs=_row_spec(3 * _OUT),
    out_shape=jax.ShapeDtypeStruct((_N, 3 * _OUT), _F32),
)


def kernel(x, edge_index, W0_0, W0_1, W0_2, W1_0, W1_1, W1_2):
  src = edge_index[0]
  dst = edge_index[1]
  pad = _E_PAD - _E
  src_p = jnp.concatenate([src, jnp.zeros((pad,), jnp.int32)])
  dst_p = jnp.concatenate([dst, jnp.full((pad,), _N, jnp.int32)])

  z16 = jnp.zeros((_N_PAD, 16), _F32)
  z128 = jnp.zeros((_N_PAD, 128), _F32)
  z64 = jnp.zeros((_N_PAD, 64), _F32)
  z80 = jnp.zeros((_N_PAD, 80), _F32)
  z48 = jnp.zeros((_N_PAD, 48), _F32)

  degacc = _deg(dst_p, z16)

  W0c = jnp.concatenate([W0_0, W0_1, W0_2], axis=1)
  p0, g, norm = _prep0(degacc, x, W0c)

  s1 = _prop(128, g, src_p, dst_p, z128)
  at1, mid = _comb0(s1, norm)
  s2 = _prop(64, mid, src_p, dst_p, z64)

  W1c = jnp.concatenate([W1_0, W1_1, W1_2], axis=1)
  q0, u = _hid(p0, at1, s2, norm, W1c)

  s3 = _prop(80, u, src_p, dst_p, z80)
  au1, mid2 = _comb1(s3, norm)
  s4 = _prop(48, mid2, src_p, dst_p, z48)

  return _fin(q0, au1, s4, norm)


# speedup vs baseline: 1.2969x; 1.1748x over previous
import functools

import jax
import jax.numpy as jnp
from jax import lax
from jax.experimental import pallas as pl
from jax.experimental.pallas import tpu as pltpu
from jax.experimental.pallas import tpu_sc as plsc

_N = 10000
_E = 320000
_NCORES = 2
_NSUB = 16
_NW = _NCORES * _NSUB
_E_PAD = 327680
_EPW0 = 18432
_EPW1 = 2048
_CFG = {
    16: (256, 4, (1, 1)),
    48: (256, 4, (1, 1)),
    64: (256, 4, (3, 1)),
    80: (256, 2, (1, 1)),
    128: (64, 4, (3, 1)),
}
_N_PAD = 10112
_ZROWS = _N_PAD // _NSUB

_HID = 64
_OUT = 40
_F32 = jnp.float32


def _make_prop(d):
  mesh = plsc.VectorSubcoreMesh(core_axis_name="c", subcore_axis_name="s",
                                num_cores=_NCORES, num_subcores=_NSUB)
  k, nbuf, rounds = _CFG[d]
  ch = (_EPW0 // k, _EPW1 // k)
  segs = (ch[0] // rounds[0], ch[1] // rounds[1])
  seg_max = max(segs)
  for s in segs:
    assert s % nbuf == 0

  def body(h_hbm, src_hbm, dst_hbm, zero_hbm, out_hbm,
           src_i, dst_i, rows_v, acc, gsems, ssems):
    cid = lax.axis_index("c")
    sid = lax.axis_index("s")
    pltpu.sync_copy(zero_hbm.at[pl.ds(sid * _ZROWS, _ZROWS)],
                    acc.at[pl.ds(sid * _ZROWS, _ZROWS)])
    plsc.subcore_barrier()

    def run(base_chunk, n_rounds, seg):
      for r in range(n_rounds):
        roff = base_chunk + r * seg
        pltpu.sync_copy(src_hbm.at[pl.ds(roff, seg)],
                        src_i.at[pl.ds(0, seg)])
        pltpu.sync_copy(dst_hbm.at[pl.ds(roff, seg)],
                        dst_i.at[pl.ds(0, seg)])
        for b in range(nbuf - 1):
          pltpu.async_copy(h_hbm.at[src_i.at[b]], rows_v.at[b], gsems.at[b])

        def step(g, carry):
          for b in range(nbuf):
            j = nbuf * g + b
            bn = (b + nbuf - 1) % nbuf
            pltpu.make_async_copy(h_hbm.at[src_i.at[j]], rows_v.at[b],
                                  gsems.at[b]).wait()
            pltpu.async_copy(rows_v.at[b], acc.at[dst_i.at[j]], ssems.at[b],
                             add=True)

            @pl.when(j >= 1)
            def _():
              pltpu.make_async_copy(rows_v.at[bn],
                                    acc.at[dst_i.at[jnp.maximum(j - 1, 0)]],
                                    ssems.at[bn]).wait()

            @pl.when(j + nbuf - 1 < seg)
            def _():
              pltpu.async_copy(h_hbm.at[src_i.at[j + nbuf - 1]],
                               rows_v.at[bn], gsems.at[bn])
          return carry

        lax.fori_loop(0, seg // nbuf, step, 0)
        pltpu.make_async_copy(rows_v.at[(seg - 1) % nbuf],
                              acc.at[dst_i.at[seg - 1]],
                              ssems.at[(seg - 1) % nbuf]).wait()

    @pl.when(cid == 0)
    def _():
      run(sid * ch[0], rounds[0], segs[0])

    @pl.when(cid == 1)
    def _():
      run(_NSUB * ch[0] + sid * ch[1], rounds[1], segs[1])

    plsc.subcore_barrier()
    pltpu.sync_copy(acc.at[pl.ds(sid * _ZROWS, _ZROWS)],
                    out_hbm.at[cid, pl.ds(sid * _ZROWS, _ZROWS)])

  return pl.kernel(
      body,
      out_type=jax.ShapeDtypeStruct((_NCORES, _N_PAD, d), _F32),
      mesh=mesh,
      compiler_params=pltpu.CompilerParams(use_tc_tiling_on_sc=False),
      scratch_types=[
          pltpu.VMEM((seg_max, k), jnp.int32),
          pltpu.VMEM((seg_max, k), jnp.int32),
          pltpu.VMEM((nbuf, k, d), _F32),
          pltpu.VMEM_SHARED((_N_PAD, d), _F32),
          pltpu.SemaphoreType.DMA((nbuf,)),
          pltpu.SemaphoreType.DMA((nbuf,)),
      ],
  )


def _make_deg():
  d, k, nbuf = 16, 256, 4
  ch = (_EPW0 // k, _EPW1 // k)
  mesh = plsc.VectorSubcoreMesh(core_axis_name="c", subcore_axis_name="s",
                                num_cores=_NCORES, num_subcores=_NSUB)

  def body(ones_hbm, dst_hbm, zero_hbm, out_hbm, dst_i, ones_v, acc,
           gsem, ssems):
    cid = lax.axis_index("c")
    sid = lax.axis_index("s")
    pltpu.sync_copy(zero_hbm.at[pl.ds(sid * _ZROWS, _ZROWS)],
                    acc.at[pl.ds(sid * _ZROWS, _ZROWS)])
    pltpu.async_copy(ones_hbm, ones_v, gsem).wait()
    plsc.subcore_barrier()

    def run(base_chunk, chunks):
      pltpu.sync_copy(dst_hbm.at[pl.ds(base_chunk, chunks)],
                      dst_i.at[pl.ds(0, chunks)])

      def step(g, carry):
        for b in range(nbuf):
          j = nbuf * g + b

          @pl.when(j >= nbuf)
          def _():
            pltpu.make_async_copy(
                ones_v, acc.at[dst_i.at[jnp.maximum(j - nbuf, 0)]],
                ssems.at[b]).wait()

          pltpu.async_copy(ones_v, acc.at[dst_i.at[j]], ssems.at[b], add=True)
        return carry

      lax.fori_loop(0, chunks // nbuf, step, 0)
      for b in range(nbuf):
        pltpu.make_async_copy(ones_v, acc.at[dst_i.at[chunks - nbuf + b]],
                              ssems.at[b]).wait()

    @pl.when(cid == 0)
    def _():
      run(sid * ch[0], ch[0])

    @pl.when(cid == 1)
    def _():
      run(_NSUB * ch[0] + sid * ch[1], ch[1])

    plsc.subcore_barrier()
    pltpu.sync_copy(acc.at[pl.ds(sid * _ZROWS, _ZROWS)],
                    out_hbm.at[cid, pl.ds(sid * _ZROWS, _ZROWS)])

  return pl.kernel(
      body,
      out_type=jax.ShapeDtypeStruct((_NCORES, _N_PAD, d), _F32),
      mesh=mesh,
      compiler_params=pltpu.CompilerParams(use_tc_tiling_on_sc=False),
      scratch_types=[
          pltpu.VMEM((max(ch), k), jnp.int32),
          pltpu.VMEM((k, d), _F32),
          pltpu.VMEM_SHARED((_N_PAD, d), _F32),
          pltpu.SemaphoreType.DMA,
          pltpu.SemaphoreType.DMA((nbuf,)),
      ],
  )


_prop_cache = {}


def _deg(dst_flat, zeros):
  if "deg" not in _prop_cache:
    _prop_cache["deg"] = _make_deg()
  k = 256
  ones = jnp.ones((k, 16), _F32)
  dst2 = dst_flat.reshape(_E_PAD // k, k)
  return _prop_cache["deg"](ones, dst2, zeros)


def _prop(d, h, src_flat, dst_flat, zeros):
  if d not in _prop_cache:
    _prop_cache[d] = _make_prop(d)
  k = _CFG[d][0]
  src2 = src_flat.reshape(_E_PAD // k, k)
  dst2 = dst_flat.reshape(_E_PAD // k, k)
  return _prop_cache[d](h, src2, dst2, zeros)


_BM = 512
_G = (_N + _BM - 1) // _BM


def _row_spec(d):
  return pl.BlockSpec((_BM, d), lambda i: (i, 0))


def _pair_spec(d):
  return pl.BlockSpec((_NCORES, _BM, d), lambda i: (0, i, 0))


def _full_spec(r, c):
  return pl.BlockSpec((r, c), lambda i: (0, 0))


def _prep0_body(deg_ref, x_ref, w_ref, p0_ref, g_ref, n_ref):
  deg = deg_ref[0, :, 0] + deg_ref[1, :, 0]
  norm = jnp.where(deg > 0, lax.rsqrt(jnp.maximum(deg, 1e-12)), 0.0)
  n = norm[:, None]
  n_ref[...] = n
  h = jnp.dot(x_ref[...], w_ref[...], preferred_element_type=_F32)
  p0_ref[...] = h[:, :_HID]
  g_ref[...] = h[:, _HID:] * n


_prep0 = pl.pallas_call(
    _prep0_body,
    grid=(_G,),
    in_specs=[_pair_spec(16), _row_spec(128), _full_spec(128, 3 * _HID)],
    out_specs=[_row_spec(_HID), _row_spec(2 * _HID), _row_spec(1)],
    out_shape=[
        jax.ShapeDtypeStruct((_N, _HID), _F32),
        jax.ShapeDtypeStruct((_N, 2 * _HID), _F32),
        jax.ShapeDtypeStruct((_N, 1), _F32),
    ],
)


def _comb_body(split, pad, s_ref, n_ref, a_ref, b_ref):
  s = s_ref[0] + s_ref[1]
  n = n_ref[...]
  a_ref[...] = s[:, :split] * n
  b = s[:, split:] * (n * n)
  if pad:
    b = jnp.concatenate([b, jnp.zeros((_BM, pad), _F32)], axis=1)
  b_ref[...] = b


def _make_comb(d, split, pad):
  return pl.pallas_call(
      functools.partial(_comb_body, split, pad),
      grid=(_G,),
      in_specs=[_pair_spec(d), _row_spec(1)],
      out_specs=[_row_spec(split), _row_spec(d - split + pad)],
      out_shape=[
          jax.ShapeDtypeStruct((_N, split), _F32),
          jax.ShapeDtypeStruct((_N, d - split + pad), _F32),
      ],
  )


_comb0 = _make_comb(2 * _HID, _HID, 0)
_comb1 = _make_comb(2 * _OUT, _OUT, 8)


def _hid_body(p0_ref, a1_ref, s2_ref, n_ref, w_ref, q0_ref, u_ref):
  n = n_ref[...]
  a2 = (s2_ref[0] + s2_ref[1]) * n
  hidden = jnp.concatenate([p0_ref[...], a1_ref[...], a2], axis=1)
  hidden = jnp.maximum(hidden, 0.0)
  h1 = jnp.dot(hidden, w_ref[...], preferred_element_type=_F32)
  q0_ref[...] = h1[:, :_OUT]
  u_ref[...] = h1[:, _OUT:] * n


_hid = pl.pallas_call(
    _hid_body,
    grid=(_G,),
    in_specs=[_row_spec(_HID), _row_spec(_HID), _pair_spec(_HID),
              _row_spec(1), _full_spec(3 * _HID, 3 * _OUT)],
    out_specs=[_row_spec(_OUT), _row_spec(2 * _OUT)],
    out_shape=[
        jax.ShapeDtypeStruct((_N, _OUT), _F32),
        jax.ShapeDtypeStruct((_N, 2 * _OUT), _F32),
    ],
)


def _fin_body(q0_ref, au1_ref, s4_ref, n_ref, o_ref):
  n = n_ref[...]
  a2u2 = (s4_ref[0] + s4_ref[1])[:, :_OUT] * n
  z = jnp.concatenate([q0_ref[...], au1_ref[...], a2u2], axis=1)
  m = jnp.max(z, axis=1, keepdims=True)
  lse = jnp.log(jnp.sum(jnp.exp(z - m), axis=1, keepdims=True)) + m
  o_ref[...] = z - lse


_fin = pl.pallas_call(
    _fin_body,
    grid=(_G,),
    in_specs=[_row_spec(_OUT), _row_spec(_OUT), _pair_spec(48), _row_spec(1)],
    out_specs=_row_spec(3 * _OUT),
    out_shape=jax.ShapeDtypeStruct((_N, 3 * _OUT), _F32),
)


def kernel(x, edge_index, W0_0, W0_1, W0_2, W1_0, W1_1, W1_2):
  src = edge_index[0]
  dst = edge_index[1]
  pad = _E_PAD - _E
  src_p = jnp.concatenate([src, jnp.zeros((pad,), jnp.int32)])
  dst_p = jnp.concatenate([dst, jnp.full((pad,), _N, jnp.int32)])

  z16 = jnp.zeros((_N_PAD, 16), _F32)
  z128 = jnp.zeros((_N_PAD, 128), _F32)
  z64 = jnp.zeros((_N_PAD, 64), _F32)
  z80 = jnp.zeros((_N_PAD, 80), _F32)
  z48 = jnp.zeros((_N_PAD, 48), _F32)

  degacc = _deg(dst_p, z16)

  W0c = jnp.concatenate([W0_0, W0_1, W0_2], axis=1)
  p0, g, norm = _prep0(degacc, x, W0c)

  s1 = _prop(128, g, src_p, dst_p, z128)
  at1, mid = _comb0(s1, norm)
  s2 = _prop(64, mid, src_p, dst_p, z64)

  W1c = jnp.concatenate([W1_0, W1_1, W1_2], axis=1)
  q0, u = _hid(p0, at1, s2, norm, W1c)

  s3 = _prop(80, u, src_p, dst_p, z80)
  au1, mid2 = _comb1(s3, norm)
  s4 = _prop(48, mid2, src_p, dst_p, z48)

  return _fin(q0, au1, s4, norm)


# speedup vs baseline: 1.3054x; 1.0065x over previous
import functools

import jax
import jax.numpy as jnp
from jax import lax
from jax.experimental import pallas as pl
from jax.experimental.pallas import tpu as pltpu
from jax.experimental.pallas import tpu_sc as plsc

_N = 10000
_E = 320000
_NCORES = 2
_NSUB = 16
_NW = _NCORES * _NSUB
_E_PAD = 327680
_EPW0 = 19456
_EPW1 = 1024
_CFG = {
    16: (256, 4, (1, 1)),
    48: (256, 4, (1, 1)),
    64: (256, 2, (1, 1)),
    80: (256, 2, (1, 1)),
    128: (64, 4, (4, 1)),
}
_N_PAD = 10112
_ZROWS = _N_PAD // _NSUB

_HID = 64
_OUT = 40
_F32 = jnp.float32


def _make_prop(d):
  mesh = plsc.VectorSubcoreMesh(core_axis_name="c", subcore_axis_name="s",
                                num_cores=_NCORES, num_subcores=_NSUB)
  k, nbuf, rounds = _CFG[d]
  ch = (_EPW0 // k, _EPW1 // k)
  segs = (ch[0] // rounds[0], ch[1] // rounds[1])
  seg_max = max(segs)
  for s in segs:
    assert s % nbuf == 0

  def body(h_hbm, src_hbm, dst_hbm, zero_hbm, out_hbm,
           src_i, dst_i, rows_v, acc, gsems, ssems):
    cid = lax.axis_index("c")
    sid = lax.axis_index("s")
    pltpu.sync_copy(zero_hbm.at[pl.ds(sid * _ZROWS, _ZROWS)],
                    acc.at[pl.ds(sid * _ZROWS, _ZROWS)])
    plsc.subcore_barrier()

    def run(base_chunk, n_rounds, seg):
      for r in range(n_rounds):
        roff = base_chunk + r * seg
        pltpu.sync_copy(src_hbm.at[pl.ds(roff, seg)],
                        src_i.at[pl.ds(0, seg)])
        pltpu.sync_copy(dst_hbm.at[pl.ds(roff, seg)],
                        dst_i.at[pl.ds(0, seg)])
        for b in range(nbuf - 1):
          pltpu.async_copy(h_hbm.at[src_i.at[b]], rows_v.at[b], gsems.at[b])

        def step(g, carry):
          for b in range(nbuf):
            j = nbuf * g + b
            bn = (b + nbuf - 1) % nbuf
            pltpu.make_async_copy(h_hbm.at[src_i.at[j]], rows_v.at[b],
                                  gsems.at[b]).wait()
            pltpu.async_copy(rows_v.at[b], acc.at[dst_i.at[j]], ssems.at[b],
                             add=True)

            @pl.when(j >= 1)
            def _():
              pltpu.make_async_copy(rows_v.at[bn],
                                    acc.at[dst_i.at[jnp.maximum(j - 1, 0)]],
                                    ssems.at[bn]).wait()

            @pl.when(j + nbuf - 1 < seg)
            def _():
              pltpu.async_copy(h_hbm.at[src_i.at[j + nbuf - 1]],
                               rows_v.at[bn], gsems.at[bn])
          return carry

        lax.fori_loop(0, seg // nbuf, step, 0)
        pltpu.make_async_copy(rows_v.at[(seg - 1) % nbuf],
                              acc.at[dst_i.at[seg - 1]],
                              ssems.at[(seg - 1) % nbuf]).wait()

    @pl.when(cid == 0)
    def _():
      run(sid * ch[0], rounds[0], segs[0])

    @pl.when(cid == 1)
    def _():
      run(_NSUB * ch[0] + sid * ch[1], rounds[1], segs[1])

    plsc.subcore_barrier()
    pltpu.sync_copy(acc.at[pl.ds(sid * _ZROWS, _ZROWS)],
                    out_hbm.at[cid, pl.ds(sid * _ZROWS, _ZROWS)])

  return pl.kernel(
      body,
      out_type=jax.ShapeDtypeStruct((_NCORES, _N_PAD, d), _F32),
      mesh=mesh,
      compiler_params=pltpu.CompilerParams(use_tc_tiling_on_sc=False),
      scratch_types=[
          pltpu.VMEM((seg_max, k), jnp.int32),
          pltpu.VMEM((seg_max, k), jnp.int32),
          pltpu.VMEM((nbuf, k, d), _F32),
          pltpu.VMEM_SHARED((_N_PAD, d), _F32),
          pltpu.SemaphoreType.DMA((nbuf,)),
          pltpu.SemaphoreType.DMA((nbuf,)),
      ],
  )


def _make_deg():
  d, k, nbuf = 16, 256, 4
  ch = (_EPW0 // k, _EPW1 // k)
  mesh = plsc.VectorSubcoreMesh(core_axis_name="c", subcore_axis_name="s",
                                num_cores=_NCORES, num_subcores=_NSUB)

  def body(ones_hbm, dst_hbm, zero_hbm, out_hbm, dst_i, ones_v, acc,
           gsem, ssems):
    cid = lax.axis_index("c")
    sid = lax.axis_index("s")
    pltpu.sync_copy(zero_hbm.at[pl.ds(sid * _ZROWS, _ZROWS)],
                    acc.at[pl.ds(sid * _ZROWS, _ZROWS)])
    pltpu.async_copy(ones_hbm, ones_v, gsem).wait()
    plsc.subcore_barrier()

    def run(base_chunk, chunks):
      pltpu.sync_copy(dst_hbm.at[pl.ds(base_chunk, chunks)],
                      dst_i.at[pl.ds(0, chunks)])

      def step(g, carry):
        for b in range(nbuf):
          j = nbuf * g + b

          @pl.when(j >= nbuf)
          def _():
            pltpu.make_async_copy(
                ones_v, acc.at[dst_i.at[jnp.maximum(j - nbuf, 0)]],
                ssems.at[b]).wait()

          pltpu.async_copy(ones_v, acc.at[dst_i.at[j]], ssems.at[b], add=True)
        return carry

      lax.fori_loop(0, chunks // nbuf, step, 0)
      for b in range(nbuf):
        pltpu.make_async_copy(ones_v, acc.at[dst_i.at[chunks - nbuf + b]],
                              ssems.at[b]).wait()

    @pl.when(cid == 0)
    def _():
      run(sid * ch[0], ch[0])

    @pl.when(cid == 1)
    def _():
      run(_NSUB * ch[0] + sid * ch[1], ch[1])

    plsc.subcore_barrier()
    pltpu.sync_copy(acc.at[pl.ds(sid * _ZROWS, _ZROWS)],
                    out_hbm.at[cid, pl.ds(sid * _ZROWS, _ZROWS)])

  return pl.kernel(
      body,
      out_type=jax.ShapeDtypeStruct((_NCORES, _N_PAD, d), _F32),
      mesh=mesh,
      compiler_params=pltpu.CompilerParams(use_tc_tiling_on_sc=False),
      scratch_types=[
          pltpu.VMEM((max(ch), k), jnp.int32),
          pltpu.VMEM((k, d), _F32),
          pltpu.VMEM_SHARED((_N_PAD, d), _F32),
          pltpu.SemaphoreType.DMA,
          pltpu.SemaphoreType.DMA((nbuf,)),
      ],
  )


_prop_cache = {}


def _deg(dst_flat, zeros):
  if "deg" not in _prop_cache:
    _prop_cache["deg"] = _make_deg()
  k = 256
  ones = jnp.ones((k, 16), _F32)
  dst2 = dst_flat.reshape(_E_PAD // k, k)
  return _prop_cache["deg"](ones, dst2, zeros)


def _prop(d, h, src_flat, dst_flat, zeros):
  if d not in _prop_cache:
    _prop_cache[d] = _make_prop(d)
  k = _CFG[d][0]
  src2 = src_flat.reshape(_E_PAD // k, k)
  dst2 = dst_flat.reshape(_E_PAD // k, k)
  return _prop_cache[d](h, src2, dst2, zeros)


_BM = 512
_G = (_N + _BM - 1) // _BM


def _row_spec(d):
  return pl.BlockSpec((_BM, d), lambda i: (i, 0))


def _pair_spec(d):
  return pl.BlockSpec((_NCORES, _BM, d), lambda i: (0, i, 0))


def _full_spec(r, c):
  return pl.BlockSpec((r, c), lambda i: (0, 0))


def _prep0_body(deg_ref, x_ref, w_ref, p0_ref, g_ref, n_ref):
  deg = deg_ref[0, :, 0] + deg_ref[1, :, 0]
  norm = jnp.where(deg > 0, lax.rsqrt(jnp.maximum(deg, 1e-12)), 0.0)
  n = norm[:, None]
  n_ref[...] = n
  h = jnp.dot(x_ref[...], w_ref[...], preferred_element_type=_F32)
  p0_ref[...] = h[:, :_HID]
  g_ref[...] = h[:, _HID:] * n


_prep0 = pl.pallas_call(
    _prep0_body,
    grid=(_G,),
    in_specs=[_pair_spec(16), _row_spec(128), _full_spec(128, 3 * _HID)],
    out_specs=[_row_spec(_HID), _row_spec(2 * _HID), _row_spec(1)],
    out_shape=[
        jax.ShapeDtypeStruct((_N, _HID), _F32),
        jax.ShapeDtypeStruct((_N, 2 * _HID), _F32),
        jax.ShapeDtypeStruct((_N, 1), _F32),
    ],
)


def _comb_body(split, pad, s_ref, n_ref, a_ref, b_ref):
  s = s_ref[0] + s_ref[1]
  n = n_ref[...]
  a_ref[...] = s[:, :split] * n
  b = s[:, split:] * (n * n)
  if pad:
    b = jnp.concatenate([b, jnp.zeros((_BM, pad), _F32)], axis=1)
  b_ref[...] = b


def _make_comb(d, split, pad):
  return pl.pallas_call(
      functools.partial(_comb_body, split, pad),
      grid=(_G,),
      in_specs=[_pair_spec(d), _row_spec(1)],
      out_specs=[_row_spec(split), _row_spec(d - split + pad)],
      out_shape=[
          jax.ShapeDtypeStruct((_N, split), _F32),
          jax.ShapeDtypeStruct((_N, d - split + pad), _F32),
      ],
  )


_comb0 = _make_comb(2 * _HID, _HID, 0)
_comb1 = _make_comb(2 * _OUT, _OUT, 8)


def _hid_body(p0_ref, a1_ref, s2_ref, n_ref, w_ref, q0_ref, u_ref):
  n = n_ref[...]
  a2 = (s2_ref[0] + s2_ref[1]) * n
  hidden = jnp.concatenate([p0_ref[...], a1_ref[...], a2], axis=1)
  hidden = jnp.maximum(hidden, 0.0)
  h1 = jnp.dot(hidden, w_ref[...], preferred_element_type=_F32)
  q0_ref[...] = h1[:, :_OUT]
  u_ref[...] = h1[:, _OUT:] * n


_hid = pl.pallas_call(
    _hid_body,
    grid=(_G,),
    in_specs=[_row_spec(_HID), _row_spec(_HID), _pair_spec(_HID),
              _row_spec(1), _full_spec(3 * _HID, 3 * _OUT)],
    out_specs=[_row_spec(_OUT), _row_spec(2 * _OUT)],
    out_shape=[
        jax.ShapeDtypeStruct((_N, _OUT), _F32),
        jax.ShapeDtypeStruct((_N, 2 * _OUT), _F32),
    ],
)


def _fin_body(q0_ref, au1_ref, s4_ref, n_ref, o_ref):
  n = n_ref[...]
  a2u2 = (s4_ref[0] + s4_ref[1])[:, :_OUT] * n
  z = jnp.concatenate([q0_ref[...], au1_ref[...], a2u2], axis=1)
  m = jnp.max(z, axis=1, keepdims=True)
  lse = jnp.log(jnp.sum(jnp.exp(z - m), axis=1, keepdims=True)) + m
  o_ref[...] = z - lse


_fin = pl.pallas_call(
    _fin_body,
    grid=(_G,),
    in_specs=[_row_spec(_OUT), _row_spec(_OUT), _pair_spec(48), _row_spec(1)],
    out_specs=_row_spec(3 * _OUT),
    out_shape=jax.ShapeDtypeStruct((_N, 3 * _OUT), _F32),
)


def kernel(x, edge_index, W0_0, W0_1, W0_2, W1_0, W1_1, W1_2):
  src = edge_index[0]
  dst = edge_index[1]
  pad = _E_PAD - _E
  src_p = jnp.concatenate([src, jnp.zeros((pad,), jnp.int32)])
  dst_p = jnp.concatenate([dst, jnp.full((pad,), _N, jnp.int32)])

  z16 = jnp.zeros((_N_PAD, 16), _F32)
  z128 = jnp.zeros((_N_PAD, 128), _F32)
  z64 = jnp.zeros((_N_PAD, 64), _F32)
  z80 = jnp.zeros((_N_PAD, 80), _F32)
  z48 = jnp.zeros((_N_PAD, 48), _F32)

  degacc = _deg(dst_p, z16)

  W0c = jnp.concatenate([W0_0, W0_1, W0_2], axis=1)
  p0, g, norm = _prep0(degacc, x, W0c)

  s1 = _prop(128, g, src_p, dst_p, z128)
  at1, mid = _comb0(s1, norm)
  s2 = _prop(64, mid, src_p, dst_p, z64)

  W1c = jnp.concatenate([W1_0, W1_1, W1_2], axis=1)
  q0, u = _hid(p0, at1, s2, norm, W1c)

  s3 = _prop(80, u, src_p, dst_p, z80)
  au1, mid2 = _comb1(s3, norm)
  s4 = _prop(48, mid2, src_p, dst_p, z48)

  return _fin(q0, au1, s4, norm)
